# baseline (speedup 1.0000x reference)
import functools

import jax
import jax.numpy as jnp
from jax import lax
from jax.experimental import pallas as pl
from jax.experimental.pallas import tpu as pltpu
from jax.experimental.pallas import tpu_sc as plsc

B = 16384
D = 32
V = 1000
L = 20
_G = 128
_RPG = _G // D

_NC = 2
_NS = 16
_NW = _NC * _NS
_BPW = B // _NW
_CHUNK = 128
_NCHUNK = _BPW // _CHUNK


def _sc_gather_body(user_table, uids, item_table, iids, ue_out, ie_out,
                    idx, rows, sem):
  wid = lax.axis_index("s") * _NC + lax.axis_index("c")
  base = wid * _BPW
  row0 = wid * _NCHUNK
  for table, ids, out in ((user_table, uids, ue_out),
                          (item_table, iids, ie_out)):
    pltpu.sync_copy(ids.at[pl.ds(row0, _NCHUNK)], idx)
    copies = []
    for j in range(_NCHUNK):
      copies.append(pltpu.async_copy(
          table.at[idx.at[j]], rows.at[pl.ds(j * _CHUNK, _CHUNK)], sem))
    for c in copies:
      c.wait()
    pltpu.sync_copy(rows, out.at[pl.ds(base, _BPW)])


@functools.partial(
    pl.kernel,
    out_type=(jax.ShapeDtypeStruct((B, _G), jnp.float32),
              jax.ShapeDtypeStruct((B, _G), jnp.float32)),
    mesh=plsc.VectorSubcoreMesh(core_axis_name="c", subcore_axis_name="s"),
    scratch_types=[
        pltpu.VMEM((_NCHUNK, _CHUNK), jnp.int32),
        pltpu.VMEM((_BPW, _G), jnp.float32),
        pltpu.SemaphoreType.DMA,
    ],
)
def _sc_gather(*args):
  _sc_gather_body(*args)


_TB = 1024


def _tc_body(lens_ref, uoff_ref, ioff_ref, ue_ref, ie_ref, wext_ref,
             logits_ref, mask_ref):
  col = lax.broadcasted_iota(jnp.int32, (_TB, _G), 1) // D
  mu = (col == uoff_ref[...]).astype(jnp.float32)
  mi = (col == ioff_ref[...]).astype(jnp.float32)
  e = jnp.concatenate([ue_ref[...] * mu, ie_ref[...] * mi], axis=1)
  logits_ref[...] = lax.dot_general(
      e, wext_ref[...], (((1,), (0,)), ((), ())),
      preferred_element_type=jnp.float32)
  io = lax.broadcasted_iota(jnp.int32, (_TB, L), 1)
  mask_ref[...] = io >= lens_ref[...]


_tc_call = pl.pallas_call(
    _tc_body,
    grid=(B // _TB,),
    in_specs=[
        pl.BlockSpec((_TB, 1), lambda i: (i, 0)),
        pl.BlockSpec((_TB, 1), lambda i: (i, 0)),
        pl.BlockSpec((_TB, 1), lambda i: (i, 0)),
        pl.BlockSpec((_TB, _G), lambda i: (i, 0)),
        pl.BlockSpec((_TB, _G), lambda i: (i, 0)),
        pl.BlockSpec((2 * _G, V), lambda i: (0, 0)),
    ],
    out_specs=[
        pl.BlockSpec((_TB, V), lambda i: (i, 0)),
        pl.BlockSpec((_TB, L), lambda i: (i, 0)),
    ],
    out_shape=[
        jax.ShapeDtypeStruct((B, V), jnp.float32),
        jax.ShapeDtypeStruct((B, L), jnp.bool_),
    ],
)


def kernel(pos_attr_set, pos_attr_lens, neg_attr_set, neg_attr_lens,
           neg_attr_set_num, user_ids, item_ids, _, user_table, item_table,
           W_user, W_item):
  uids = user_ids.astype(jnp.int32)
  iids = item_ids.astype(jnp.int32)
  uid_g = (uids >> 2).reshape(B // _CHUNK, _CHUNK)
  iid_g = (iids >> 2).reshape(B // _CHUNK, _CHUNK)
  ut = user_table.reshape(-1, _G)
  it = item_table.reshape(-1, _G)
  ue, ie = _sc_gather(ut, uid_g, it, iid_g)
  wext = jnp.concatenate(
      [jnp.tile(W_user.T, (_RPG, 1)), jnp.tile(W_item.T, (_RPG, 1))], axis=0)
  logits, mask = _tc_call(
      pos_attr_lens.astype(jnp.int32).reshape(B, 1),
      (uids & 3).reshape(B, 1), (iids & 3).reshape(B, 1),
      ue, ie, wext)
  return (logits, mask)
